# baseline (speedup 1.0000x reference)
import functools

import jax
import jax.numpy as jnp
from jax import lax
from jax.experimental import pallas as pl
from jax.experimental.pallas import tpu as pltpu
from jax.experimental.pallas import tpu_sc as plsc

_L = 16


def _tc_body(am_ref, out_ref, *, B, AH, AW):
    am = am_ref[...]
    sel = (am >= 0).astype(jnp.float32)
    per1 = jnp.maximum(am, 0.0) + jnp.log1p(jnp.exp(-jnp.abs(am)))
    row_l = jnp.sum(per1 * sel, axis=1, keepdims=True)
    row_s = jnp.sum(sel, axis=1, keepdims=True)
    rows2 = jnp.concatenate([row_l, row_s], axis=1)
    seg = (lax.broadcasted_iota(jnp.int32, (B, B * AH), 1) // AH
           == lax.broadcasted_iota(jnp.int32, (B, B * AH), 0)
           ).astype(jnp.float32)
    out_ref[...] = lax.dot_general(seg, rows2, (((1,), (0,)), ((), ())),
                                   preferred_element_type=jnp.float32)


def _sc_body(hw_hbm, tgt_hbm, am_hbm, out_hbm,
             hw_v, tgt_v, amv, x1a, x2a, rloa, rhia, mask_v, outv, sem,
             *, B, AH, AW, N, NC, NS):
    wid = lax.axis_index("s") * NC + lax.axis_index("c")
    nw = NC * NS
    per_img = nw // B
    rows_per_tile = AH // per_img
    chunk = rows_per_tile * AW
    myimg = wid // per_img
    half = wid % per_img
    base_row = half * rows_per_tile

    am_cp = pltpu.make_async_copy(
        am_hbm.at[pl.ds(wid * chunk, chunk)], amv, sem)
    am_cp.start()
    pltpu.sync_copy(hw_hbm, hw_v)
    pltpu.sync_copy(tgt_hbm, tgt_v)

    hv = hw_v[0, :]
    wv = hw_v[1, :]
    sxv = jnp.float32(AW) / wv
    syv = jnp.float32(AH) / hv
    myf = jnp.full((_L,), myimg, jnp.int32).astype(jnp.float32)

    has_acc = jnp.float32(0.0)
    cnt_lt = jnp.zeros((_L,), jnp.int32)
    cnt_eq = jnp.zeros((_L,), jnp.int32)
    for g in range(N // _L):
        sl = pl.ds(g * _L, _L)
        imgid = tgt_v[0, sl]
        x = tgt_v[2, sl]
        y = tgt_v[3, sl]
        bw = tgt_v[4, sl]
        bh = tgt_v[5, sl]
        bx1 = wv * (x - bw * 0.5)
        by1 = hv * (y - bh * 0.5)
        bx2 = wv * (x + bw * 0.5)
        by2 = hv * (y + bh * 0.5)
        cond = (bx1 <= wv) & (by1 <= hv) & (bx2 <= wv) & (by2 <= hv)
        lx1 = bx1 * sxv
        ly1 = by1 * syv
        lx2 = bx2 * sxv
        ly2 = by2 * syv
        x1i = jnp.maximum(lx1.astype(jnp.int32), 0)
        y1i = jnp.maximum(ly1.astype(jnp.int32), 0)
        tx = lx2.astype(jnp.int32)
        cx = tx + (lx2 > tx.astype(jnp.float32)).astype(jnp.int32)
        ty = ly2.astype(jnp.int32)
        cy = ty + (ly2 > ty.astype(jnp.float32)).astype(jnp.int32)
        x2i = jnp.minimum(cx + 1, AW)
        y2i = jnp.minimum(cy + 1, AH)
        belongs = imgid == myf
        ok = belongs & cond
        rlo = jnp.clip(y1i - base_row, 0, rows_per_tile)
        rhi = jnp.clip(y2i - base_row, 0, rows_per_tile)
        rlo = jnp.where(ok, rlo, 0)
        rhi = jnp.where(ok, rhi, 0)
        x1a[sl] = x1i
        x2a[sl] = x2i
        rloa[sl] = rlo
        rhia[sl] = rhi
        has_acc = jnp.maximum(has_acc, jnp.max(belongs.astype(jnp.float32)))
        cnt_lt = cnt_lt + (imgid < myf).astype(jnp.int32)
        cnt_eq = cnt_eq + belongs.astype(jnp.int32)
    start = jnp.sum(cnt_lt)
    end = start + jnp.sum(cnt_eq)

    zero = jnp.zeros((_L,), jnp.float32)

    def zbody(i, c):
        mask_v[pl.ds(i * _L, _L)] = zero
        return c
    lax.fori_loop(0, chunk // _L, zbody, 0)

    coli = [lax.iota(jnp.int32, _L) + cc * _L for cc in range(AW // _L)]

    def box_body(b, c):
        rlo = rloa[pl.ds(b, _L)][0]
        rhi = rhia[pl.ds(b, _L)][0]

        @pl.when(rhi > rlo)
        def _():
            x1b = jnp.full((_L,), x1a[pl.ds(b, _L)][0], jnp.int32)
            x2b = jnp.full((_L,), x2a[pl.ds(b, _L)][0], jnp.int32)
            incs = [(ci >= x1b) & (ci < x2b) for ci in coli]

            def row_body(r, c2):
                rb = r * AW
                for cc in range(AW // _L):
                    sl2 = pl.ds(rb + cc * _L, _L)
                    mask_v[sl2] = jnp.where(incs[cc], 1.0, mask_v[sl2])
                return c2
            lax.fori_loop(rlo, rhi, row_body, 0)
        return c
    lax.fori_loop(start, end, box_body, 0)

    am_cp.wait()

    def red(i, acc):
        sl3 = pl.ds(i * _L, _L)
        a = amv[sl3]
        m = mask_v[sl3]
        return acc + jnp.where((m > 0.0) & (a >= 0.0), a, 0.0)
    accv = lax.fori_loop(0, chunk // _L, red, jnp.zeros((_L,), jnp.float32))
    tpart = jnp.sum(accv)
    ii = lax.iota(jnp.int32, _L)
    outv[...] = jnp.where(ii == 0, tpart, jnp.where(ii == 1, has_acc, 0.0))
    pltpu.sync_copy(outv, out_hbm.at[wid])


def kernel(attention_mask, target, img_batch_shape):
    B, _, AH, AW = attention_mask.shape
    N = target.shape[0]
    if N == 0:
        return jnp.float32(0.0)
    info = plsc.get_sparse_core_info()
    NC, NS = info.num_cores, info.num_subcores
    nw = NC * NS
    per_img = nw // B
    chunk = (AH // per_img) * AW

    hw = jnp.broadcast_to(
        jnp.asarray(img_batch_shape).astype(jnp.float32)[2:4, None], (2, _L))
    tgt = jnp.transpose(target.astype(jnp.float32))
    am2 = attention_mask.reshape(B * AH, AW)
    am_flat = attention_mask.reshape(B * AH * AW)

    tc_out = pl.pallas_call(
        functools.partial(_tc_body, B=B, AH=AH, AW=AW),
        out_shape=jax.ShapeDtypeStruct((B, 2), jnp.float32),
    )(am2)

    mesh = plsc.VectorSubcoreMesh(core_axis_name="c", subcore_axis_name="s")
    sc = pl.kernel(
        functools.partial(_sc_body, B=B, AH=AH, AW=AW, N=N, NC=NC, NS=NS),
        mesh=mesh,
        compiler_params=pltpu.CompilerParams(needs_layout_passes=False),
        out_type=jax.ShapeDtypeStruct((nw, _L), jnp.float32),
        scratch_types=[
            pltpu.VMEM((2, _L), jnp.float32),
            pltpu.VMEM((6, N), jnp.float32),
            pltpu.VMEM((chunk,), jnp.float32),
            pltpu.VMEM((N + _L,), jnp.int32),
            pltpu.VMEM((N + _L,), jnp.int32),
            pltpu.VMEM((N + _L,), jnp.int32),
            pltpu.VMEM((N + _L,), jnp.int32),
            pltpu.VMEM((chunk,), jnp.float32),
            pltpu.VMEM((_L,), jnp.float32),
            pltpu.SemaphoreType.DMA,
        ],
    )
    sc_out = sc(hw, tgt, am_flat)
    parts = sc_out.reshape(B, per_img, _L)
    t_j = jnp.sum(parts[:, :, 0], axis=1)
    has = jnp.max(parts[:, :, 1], axis=1)
    s1 = tc_out[:, 0]
    den = tc_out[:, 1]
    return jnp.sum(jnp.where(has > 0, (s1 - t_j) / den, 0.0))
